# baseline (speedup 1.0000x reference)
import numpy as np

import jax
import jax.numpy as jnp
from jax.experimental import pallas as pl
from jax.experimental.pallas import tpu as pltpu

_M, _K, _D = 4, 8192, 32
_EPS_BOUND = 1e-06
_SCALE = np.sqrt(_K).astype(np.float32)


def _argmax_onehot(val, w_ref, code_ref, oneh_ref):
    hw = val.shape[0]
    mx = jnp.max(val, axis=1, keepdims=True)
    eq = val == mx
    eqf = jnp.where(eq, 1.0, 0.0).astype(jnp.float32)
    red = jax.lax.dot_general(
        eqf, w_ref[...], (((1,), (0,)), ((), ())),
        precision=jax.lax.Precision.HIGHEST,
        preferred_element_type=jnp.float32)
    cnt = red[:, 0:1]
    idx = red[:, 1:2]
    unique = jnp.logical_and(jnp.max(cnt) == 1.0, jnp.min(cnt) == 1.0)

    def fast(_):
        code_ref[0, 0] = idx.astype(jnp.int32)
        oneh_ref[0, 0] = eqf

    def slow(_):
        iota = jax.lax.broadcasted_iota(jnp.int32, (hw, _K), 1)
        code = jnp.min(jnp.where(eq, iota, _K), axis=1, keepdims=True)
        code_ref[0, 0] = code
        oneh_ref[0, 0] = (iota == code).astype(jnp.float32)

    jax.lax.cond(unique, fast, slow, None)


def _vq_body(temp_ref, x_ref, cb_ref, g_ref, w_ref,
             logit_ref, code_ref, oneh_ref, samp_ref, codeg_ref):
    m = pl.program_id(0)
    xv = x_ref[0, 0]
    cb = cb_ref[0]
    g = g_ref[0, 0]

    inter = jnp.dot(xv, cb, preferred_element_type=jnp.float32)
    x2 = jnp.sum(xv * xv, axis=1, keepdims=True)
    c2 = jnp.sum(cb * cb, axis=0, keepdims=True)
    dist = (x2 + c2) - 2.0 * inter
    t = jnp.maximum(temp_ref[m, 0], _EPS_BOUND)
    logit = (-dist) / _SCALE * t

    logit_ref[0, 0] = logit
    _argmax_onehot(logit, w_ref, code_ref, oneh_ref)
    _argmax_onehot(logit + g, w_ref, codeg_ref, samp_ref)


_BHW = 128


def _vq_call(xt, cbT, gumb, temp, w):
    n, M, HW, D = xt.shape
    K = cbT.shape[2]
    grid = (M, n, HW // _BHW)
    out_shapes = (
        jax.ShapeDtypeStruct((n, M, HW, K), jnp.float32),
        jax.ShapeDtypeStruct((n, M, HW, 1), jnp.int32),
        jax.ShapeDtypeStruct((n, M, HW, K), jnp.float32),
        jax.ShapeDtypeStruct((n, M, HW, K), jnp.float32),
        jax.ShapeDtypeStruct((n, M, HW, 1), jnp.int32),
    )
    big = lambda m, i, r: (i, m, r, 0)
    in_specs = [
        pl.BlockSpec(memory_space=pltpu.SMEM),
        pl.BlockSpec((1, 1, _BHW, D), big),
        pl.BlockSpec((1, D, K), lambda m, i, r: (m, 0, 0)),
        pl.BlockSpec((1, 1, _BHW, K), big),
        pl.BlockSpec((K, 128), lambda m, i, r: (0, 0)),
    ]
    out_specs = (
        pl.BlockSpec((1, 1, _BHW, K), big),
        pl.BlockSpec((1, 1, _BHW, 1), big),
        pl.BlockSpec((1, 1, _BHW, K), big),
        pl.BlockSpec((1, 1, _BHW, K), big),
        pl.BlockSpec((1, 1, _BHW, 1), big),
    )
    return pl.pallas_call(
        _vq_body, grid=grid, in_specs=in_specs, out_specs=out_specs,
        out_shape=out_shapes,
        compiler_params=pltpu.CompilerParams(
            dimension_semantics=("parallel", "parallel", "parallel")),
    )(temp, xt, cbT, gumb, w)


_GUMB_CACHE = {}


def _gumbels(n, M, h, w, K):
    shp = (n, M, h, w, K)
    if shp not in _GUMB_CACHE:
        with jax.ensure_compile_time_eval():
            eps = jnp.finfo(jnp.float32).eps
            u = jax.random.uniform(jax.random.key(42), shp, jnp.float32)
            u = jnp.clip(u, eps, 1.0 - eps)
            _GUMB_CACHE[shp] = (-jnp.log(-jnp.log(u))).reshape(n, M, h * w, K)
    return _GUMB_CACHE[shp]


def _count_iota(K):
    w = np.zeros((K, 128), np.float32)
    w[:, 0] = 1.0
    w[:, 1] = np.arange(K, dtype=np.float32)
    return jnp.asarray(w)


def kernel(x, codebook, temperature):
    n, c, h, w = x.shape
    M, K, D = codebook.shape
    hw = h * w

    gumb = _gumbels(n, M, h, w, K)

    xt = x.reshape(n, M, D, hw).transpose(0, 1, 3, 2)
    cbT = codebook.transpose(0, 2, 1)
    temp = temperature.reshape(M, 1)

    logit, code, oneh, samp, codeg = _vq_call(
        xt, cbT, gumb, temp, _count_iota(K))

    logit5 = logit.reshape(n, M, h, w, K)
    code4 = code.reshape(n, M, h, w)
    oneh5 = oneh.reshape(n, M, h, w, K)
    samp5 = samp.reshape(n, M, h, w, K)
    return (samp5, code4, oneh5, logit5)


# speedup vs baseline: 1.9991x; 1.9991x over previous
import numpy as np

import jax
import jax.numpy as jnp
from jax.experimental import pallas as pl
from jax.experimental.pallas import tpu as pltpu

_M, _K, _D = 4, 8192, 32
_EPS_BOUND = 1e-06
_SCALE = np.sqrt(_K).astype(np.float32)


def _vq_body(temp_ref, x_ref, cb_ref, g_ref,
             logit_ref, code_ref, oneh_ref, samp_ref, codeg_ref):
    m = pl.program_id(0)
    xv = x_ref[0, 0]
    cb = cb_ref[0]
    g = g_ref[0, 0]

    inter = jnp.dot(xv, cb, preferred_element_type=jnp.float32)
    x2 = jnp.sum(xv * xv, axis=1, keepdims=True)
    c2 = jnp.sum(cb * cb, axis=0, keepdims=True)
    dist = (x2 + c2) - 2.0 * inter
    t = jnp.maximum(temp_ref[m, 0], _EPS_BOUND)
    logit = (-dist) / _SCALE * t

    hw = logit.shape[0]
    iota = jax.lax.broadcasted_iota(jnp.int32, (hw, _K), 1)

    mx = jnp.max(logit, axis=1, keepdims=True)
    code = jnp.min(jnp.where(logit == mx, iota, _K), axis=1, keepdims=True)

    y = logit + g
    mxg = jnp.max(y, axis=1, keepdims=True)
    codeg = jnp.min(jnp.where(y == mxg, iota, _K), axis=1, keepdims=True)

    logit_ref[0, 0] = logit
    code_ref[0, 0] = code
    codeg_ref[0, 0] = codeg
    oneh_ref[0, 0] = (iota == code).astype(jnp.float32)
    samp_ref[0, 0] = (iota == codeg).astype(jnp.float32)


_BHW = 128


def _vq_call(xt, cbT, gumb, temp):
    n, M, HW, D = xt.shape
    K = cbT.shape[2]
    grid = (M, n, HW // _BHW)
    out_shapes = (
        jax.ShapeDtypeStruct((n, M, HW, K), jnp.float32),
        jax.ShapeDtypeStruct((n, M, HW, 1), jnp.int32),
        jax.ShapeDtypeStruct((n, M, HW, K), jnp.float32),
        jax.ShapeDtypeStruct((n, M, HW, K), jnp.float32),
        jax.ShapeDtypeStruct((n, M, HW, 1), jnp.int32),
    )
    big = lambda m, i, r: (i, m, r, 0)
    in_specs = [
        pl.BlockSpec(memory_space=pltpu.SMEM),
        pl.BlockSpec((1, 1, _BHW, D), big),
        pl.BlockSpec((1, D, K), lambda m, i, r: (m, 0, 0)),
        pl.BlockSpec((1, 1, _BHW, K), big),
    ]
    out_specs = (
        pl.BlockSpec((1, 1, _BHW, K), big),
        pl.BlockSpec((1, 1, _BHW, 1), big),
        pl.BlockSpec((1, 1, _BHW, K), big),
        pl.BlockSpec((1, 1, _BHW, K), big),
        pl.BlockSpec((1, 1, _BHW, 1), big),
    )
    return pl.pallas_call(
        _vq_body, grid=grid, in_specs=in_specs, out_specs=out_specs,
        out_shape=out_shapes,
        compiler_params=pltpu.CompilerParams(
            dimension_semantics=("parallel", "parallel", "parallel")),
    )(temp, xt, cbT, gumb)


_GUMB_CACHE = {}


def _gumbels(n, M, h, w, K):
    shp = (n, M, h, w, K)
    if shp not in _GUMB_CACHE:
        with jax.ensure_compile_time_eval():
            eps = jnp.finfo(jnp.float32).eps
            u = jax.random.uniform(jax.random.key(42), shp, jnp.float32)
            u = jnp.clip(u, eps, 1.0 - eps)
            _GUMB_CACHE[shp] = (-jnp.log(-jnp.log(u))).reshape(n, M, h * w, K)
    return _GUMB_CACHE[shp]


def kernel(x, codebook, temperature):
    n, c, h, w = x.shape
    M, K, D = codebook.shape
    hw = h * w

    gumb = _gumbels(n, M, h, w, K)

    xt = x.reshape(n, M, D, hw).transpose(0, 1, 3, 2)
    cbT = codebook.transpose(0, 2, 1)
    temp = temperature.reshape(M, 1)

    logit, code, oneh, samp, codeg = _vq_call(xt, cbT, gumb, temp)

    logit5 = logit.reshape(n, M, h, w, K)
    code4 = code.reshape(n, M, h, w)
    oneh5 = oneh.reshape(n, M, h, w, K)
    samp5 = samp.reshape(n, M, h, w, K)
    return (samp5, code4, oneh5, logit5)
